# Initial kernel scaffold; baseline (speedup 1.0000x reference)
import jax
import jax.numpy as jnp
from jax.experimental import pallas as pl


def kernel(input_tensor):
    raise NotImplementedError("write your pallas kernel here")


# speedup vs baseline: 3.8552x; 3.8552x over previous
import jax
import jax.numpy as jnp
from jax.experimental import pallas as pl
from jax.experimental.pallas import tpu as pltpu

_N_ITERS = 21
_INV_TEMP = 100.0
_M = 36


def _sinkhorn_block(x_ref, o_ref):
    x = x_ref[...]
    xt = x.T * _INV_TEMP
    bb = xt.shape[1]
    a = xt.reshape(_M, _M, bb)
    for _ in range(_N_ITERS):
        m = jnp.max(a, axis=1, keepdims=True)
        s = jnp.sum(jnp.exp(a - m), axis=1, keepdims=True)
        a = a - (jnp.log(s) + m)
        m = jnp.max(a, axis=0, keepdims=True)
        s = jnp.sum(jnp.exp(a - m), axis=0, keepdims=True)
        a = a - (jnp.log(s) + m)
    y = jnp.exp(a).reshape(_M * _M, bb)
    o_ref[...] = y.T


def kernel(input_tensor):
    b, n = input_tensor.shape
    bb = 256
    y = pl.pallas_call(
        _sinkhorn_block,
        grid=(b // bb,),
        in_specs=[pl.BlockSpec((bb, n), lambda i: (i, 0))],
        out_specs=pl.BlockSpec((bb, n), lambda i: (i, 0)),
        out_shape=jax.ShapeDtypeStruct((b, n), input_tensor.dtype),
        compiler_params=pltpu.CompilerParams(
            dimension_semantics=("parallel",),
            vmem_limit_bytes=100 * 1024 * 1024,
        ),
    )(input_tensor)
    return y.reshape(-1, _M, _M)


# speedup vs baseline: 4.8678x; 1.2627x over previous
import math

import jax
import jax.numpy as jnp
from jax.experimental import pallas as pl
from jax.experimental.pallas import tpu as pltpu

_N_ITERS = 21
_M = 36
_SCALE = 100.0 * math.log2(math.e)


def _sinkhorn_block(x_ref, o_ref):
    x = x_ref[...]
    xt = x.T * _SCALE
    s = [xt[k * _M:(k + 1) * _M, :] for k in range(_M)]

    v = None
    u = None
    for _ in range(_N_ITERS):
        u = []
        for k in range(_M):
            t = s[k] if v is None else s[k] - v
            m = jnp.max(t, axis=0, keepdims=True)
            z = jnp.sum(jnp.exp2(t - m), axis=0, keepdims=True)
            u.append(m + jnp.log2(z))
        mc = s[0] - u[0]
        for k in range(1, _M):
            mc = jnp.maximum(mc, s[k] - u[k])
        z = jnp.exp2(s[0] - u[0] - mc)
        for k in range(1, _M):
            z = z + jnp.exp2(s[k] - u[k] - mc)
        v = mc + jnp.log2(z)

    y = jnp.concatenate(
        [jnp.exp2(s[k] - u[k] - v) for k in range(_M)], axis=0)
    o_ref[...] = y.T


def kernel(input_tensor):
    b, n = input_tensor.shape
    bb = 256
    y = pl.pallas_call(
        _sinkhorn_block,
        grid=(b // bb,),
        in_specs=[pl.BlockSpec((bb, n), lambda i: (i, 0))],
        out_specs=pl.BlockSpec((bb, n), lambda i: (i, 0)),
        out_shape=jax.ShapeDtypeStruct((b, n), input_tensor.dtype),
        compiler_params=pltpu.CompilerParams(
            dimension_semantics=("parallel",),
            vmem_limit_bytes=100 * 1024 * 1024,
        ),
    )(input_tensor)
    return y.reshape(-1, _M, _M)
